# baseline (speedup 1.0000x reference)
import functools
import math

import jax
import jax.numpy as jnp
from jax import lax
from jax.experimental import pallas as pl
from jax.experimental.pallas import tpu as pltpu
from jax.experimental.pallas import tpu_sc as plsc

_NC = 80
_C_TEXT = 512
_EMBED = 128
_GCN_H = 64
_THR = 0.5
_LEVELS = [(192, 64), (384, 32), (768, 16)]
_K_RATIO = 0.005
_INTERPRET = False

_f32 = jnp.float32


def _silu(x):
    return x * jax.nn.sigmoid(x)


def _conv3(xin, xp_ref, w_ref, W, HW):
    cin = xin.shape[1]
    xp_ref[0:W + 1, :] = jnp.zeros((W + 1, cin), _f32)
    xp_ref[W + 1:W + 1 + HW, :] = xin
    xp_ref[W + 1 + HW:, :] = jnp.zeros((W + 1, cin), _f32)
    col = lax.broadcasted_iota(jnp.int32, (HW, 1), 0) & (W - 1)
    cout = w_ref.shape[3]
    acc = jnp.zeros((HW, cout), _f32)
    for dy in range(3):
        for dx in range(3):
            off = dy * W + dx
            tap = xp_ref[off:off + HW, :]
            if dx == 0:
                tap = jnp.where(col != 0, tap, 0.0)
            elif dx == 2:
                tap = jnp.where(col != W - 1, tap, 0.0)
            acc = acc + jax.lax.dot(tap, w_ref[dy, dx],
                                    preferred_element_type=_f32)
    return acc


def _trunk_body(H, W, ch, hid, c, ns,
                p1_ref, p2_ref, text_ref,
                wcv1_ref, bcv1_ref, wcv2_ref, bcv2_ref,
                wimg_ref, bimg_ref, twt_ref, tb_ref, ab_ref,
                wproj_ref, bproj_ref, wf_ref, bf_ref,
                fused_ref, idx_ref,
                xp1_ref, xp2_ref):
    HW = H * W
    p1 = p1_ref[0]
    p2 = p2_ref[0]
    hidv = _silu(jax.lax.dot(p2, wcv1_ref[...], preferred_element_type=_f32)
                 + bcv1_ref[...])
    bout = p2 + _silu(_conv3(hidv, xp1_ref, wcv2_ref, W, HW) + bcv2_ref[...])
    ie = jax.lax.dot(bout, wimg_ref[...], preferred_element_type=_f32) \
        + bimg_ref[...]
    t = jax.lax.dot(text_ref[0], twt_ref[...], preferred_element_type=_f32) \
        + tb_ref[...]
    aw = lax.dot_general(ie, t, (((1,), (1,)), ((), ())),
                         preferred_element_type=_f32)
    s = jax.nn.sigmoid(jnp.max(aw, axis=1, keepdims=True)
                       * (1.0 / math.sqrt(_EMBED)) + ab_ref[...])
    awt = lax.dot_general(t, ie, (((1,), (1,)), ((), ())),
                          preferred_element_type=_f32)
    s_row = jax.nn.sigmoid(jnp.max(awt, axis=0, keepdims=True)
                           * (1.0 / math.sqrt(_EMBED)) + ab_ref[...])
    attn = (_conv3(bout, xp2_ref, wproj_ref, W, HW) + bproj_ref[...]) * s
    fused = (jax.lax.dot(p1, wf_ref[0:ch, :], preferred_element_type=_f32)
             + jax.lax.dot(p2, wf_ref[ch:2 * ch, :],
                           preferred_element_type=_f32)
             + jax.lax.dot(bout, wf_ref[2 * ch:3 * ch, :],
                           preferred_element_type=_f32)
             + jax.lax.dot(attn, wf_ref[3 * ch:4 * ch, :],
                           preferred_element_type=_f32)
             + bf_ref[...])
    fused_ref[0] = fused
    lane = lax.broadcasted_iota(jnp.int32, (1, HW), 1)
    lane_ns = lax.broadcasted_iota(jnp.int32, (1, ns), 1)
    sc = s_row
    idxs = jnp.zeros((1, ns), jnp.int32)
    for j in range(ns):
        m = jnp.max(sc, axis=1, keepdims=True)
        cand = jnp.where(sc == m, lane, HW)
        ix = jnp.min(cand, axis=1, keepdims=True)
        idxs = idxs + jnp.where(lane_ns == j, ix, 0)
        sc = jnp.where(lane == ix, -jnp.inf, sc)
    idx_ref[0] = idxs


def _gcn_one(n, top, w1, b1, w2, b2, u_ref):
    nrm = jnp.maximum(jnp.sqrt(jnp.sum(top * top, axis=1, keepdims=True)),
                      1e-12)
    nf = top / nrm
    sim = lax.dot_general(nf, nf, (((1,), (1,)), ((), ())),
                          preferred_element_type=_f32)
    ri = lax.broadcasted_iota(jnp.int32, (n, n), 0)
    ci = lax.broadcasted_iota(jnp.int32, (n, n), 1)
    at = (sim > _THR).astype(_f32) + (ri == ci).astype(_f32)
    degr = jnp.sum(at, axis=1, keepdims=True)
    degc = jnp.sum(at, axis=0, keepdims=True)
    dr = jnp.where(degr > 0, 1.0 / jnp.sqrt(degr), 0.0)
    dc = jnp.where(degc > 0, 1.0 / jnp.sqrt(degc), 0.0)
    p = at * dr * dc
    h1 = jax.nn.relu(
        jax.lax.dot(p, jax.lax.dot(top, w1, preferred_element_type=_f32),
                    preferred_element_type=_f32) + b1)
    u_ref[...] = jax.lax.dot(
        p, jax.lax.dot(h1, w2, preferred_element_type=_f32),
        preferred_element_type=_f32) + b2


def _gcn3_body(ns_, *refs):
    for li in range(3):
        top, w1, b1, w2, b2 = refs[5 * li:5 * li + 5]
        _gcn_one(ns_[li], top[...], w1[...], b1[...], w2[...], b2[...],
                 refs[15 + li])


_SC_MESH = plsc.VectorSubcoreMesh(core_axis_name="c", subcore_axis_name="s")


def _sc_gather3(nps, cs):
    scratch = []
    for li in range(3):
        scratch += [pltpu.VMEM((nps[li],), jnp.int32),
                    pltpu.VMEM((nps[li], cs[li]), _f32),
                    pltpu.SemaphoreType.DMA]

    @functools.partial(
        pl.kernel, mesh=_SC_MESH,
        out_type=tuple(jax.ShapeDtypeStruct((nps[li], cs[li]), _f32)
                       for li in range(3)),
        scratch_types=scratch,
        compiler_params=pltpu.CompilerParams(use_tc_tiling_on_sc=False),
        interpret=_INTERPRET,
    )
    def k(f0, f1, f2, g0, g1, g2, t0, t1, t2, *scr):
        fs = (f0, f1, f2)
        gs = (g0, g1, g2)
        ts = (t0, t1, t2)
        cid = lax.axis_index("c")
        sid = lax.axis_index("s")
        for li in range(3):
            @pl.when((cid == 0) & (sid == li))
            def _(li=li):
                idx_v, rows_v, sem = scr[3 * li:3 * li + 3]
                pltpu.sync_copy(gs[li], idx_v)
                pltpu.async_copy(fs[li].at[idx_v], rows_v, sem).wait()
                pltpu.sync_copy(rows_v, ts[li])
    return k


def _scatter_t_body(ns, fused_ref, u_ref, idx_ref, out_ref, xp_ref):
    xp_ref[...] = fused_ref[0]
    for j in range(ns):
        xp_ref[pl.ds(idx_ref[0, 0, j], 1), :] = u_ref[0, j:j + 1, :]
    out_ref[0] = xp_ref[...].T


def _batch_spec(shape, space=None):
    if space is None:
        return pl.BlockSpec((1,) + tuple(shape),
                            lambda b: (b,) + (0,) * len(shape))
    return pl.BlockSpec((1,) + tuple(shape),
                        lambda b: (b,) + (0,) * len(shape),
                        memory_space=space)


def _full_spec(a):
    nd = a.ndim
    return pl.BlockSpec(a.shape, lambda b, _n=nd: (0,) * _n)


def _fold1x1(p):
    w = (p['w'][:, :, 0, 0] * p['gamma'][:, None]).T
    return w, p['beta'][None, :]


def _fold3x3(p):
    w = (p['w'] * p['gamma'][:, None, None, None]).transpose(2, 3, 1, 0)
    return w, p['beta'][None, :]


def _pad_rows(a, np_):
    n = a.shape[0]
    if np_ == n:
        return a
    pad = jnp.broadcast_to(a[-1:], (np_ - n,) + a.shape[1:])
    return jnp.concatenate([a, pad])


def _trunk(xr, text_feat, p, c, hw_side, ns):
    bsz = xr.shape[0]
    H = W = hw_side
    HW = H * W
    ch = c // 2
    hid = ch // 2

    p1 = xr[:, :, :ch]
    p2 = xr[:, :, ch:]
    wcv1, bcv1 = _fold1x1(p['cv1'])
    wcv2, bcv2 = _fold3x3(p['cv2'])
    wimg, bimg = _fold1x1(p['img_conv'])
    twt = p['text_w'].T
    tb = p['text_b'][None, :]
    ab = p['attn_bias'].reshape(1, 1)
    wproj, bproj = _fold3x3(p['proj_conv'])
    wf, bf = _fold1x1(p['fusion'])

    trunk_in = (p1, p2, text_feat, wcv1, bcv1, wcv2, bcv2, wimg, bimg,
                twt, tb, ab, wproj, bproj, wf, bf)
    in_specs = [_batch_spec((HW, ch)), _batch_spec((HW, ch)),
                _batch_spec((_NC, _C_TEXT))] + \
        [_full_spec(a) for a in trunk_in[3:]]
    fused, idx = pl.pallas_call(
        functools.partial(_trunk_body, H, W, ch, hid, c, ns),
        grid=(bsz,),
        in_specs=in_specs,
        out_specs=[_batch_spec((HW, c)), _batch_spec((1, ns))],
        out_shape=[jax.ShapeDtypeStruct((bsz, HW, c), _f32),
                   jax.ShapeDtypeStruct((bsz, 1, ns), jnp.int32)],
        scratch_shapes=[pltpu.VMEM((HW + 2 * W + 2, hid), _f32),
                        pltpu.VMEM((HW + 2 * W + 2, ch), _f32)],
        interpret=_INTERPRET,
    )(*trunk_in)
    return fused, idx


def kernel(x3, x4, x5, text_feat, params):
    bsz = x3.shape[0]
    cs = [c for c, _ in _LEVELS]
    fuseds, idxs, gidxps, nps, ns_list, n_list = [], [], [], [], [], []
    gcn_w = []
    for i, (x, (c, hw)) in enumerate(zip((x3, x4, x5), _LEVELS)):
        ns = int(hw * hw * _K_RATIO)
        HW = hw * hw
        p = params['l%d' % i]
        xr = x.reshape(bsz, c, HW).transpose(0, 2, 1)
        fused, idx = _trunk(xr, text_feat, p, c, hw, ns)
        n = bsz * ns
        np_ = -(-n // 16) * 16
        gidx = (idx.reshape(bsz, ns)
                + jnp.arange(bsz, dtype=jnp.int32)[:, None] * HW).reshape(n)
        fuseds.append(fused)
        idxs.append(idx)
        gidxps.append(_pad_rows(gidx, np_))
        nps.append(np_)
        ns_list.append(ns)
        n_list.append(n)
        gcn_w.append((p['gcn1_w'], p['gcn1_b'][None, :],
                      p['gcn2_w'], p['gcn2_b'][None, :]))

    tops = _sc_gather3(nps, cs)(
        fuseds[0].reshape(-1, cs[0]), fuseds[1].reshape(-1, cs[1]),
        fuseds[2].reshape(-1, cs[2]), *gidxps)

    gcn_in = []
    for li in range(3):
        gcn_in += [tops[li][:n_list[li]], *gcn_w[li]]
    us = pl.pallas_call(
        functools.partial(_gcn3_body, n_list),
        in_specs=[pl.BlockSpec(a.shape, functools.partial(
            lambda _n: (0,) * _n, a.ndim)) for a in gcn_in],
        out_specs=[pl.BlockSpec((n_list[li], cs[li]), lambda: (0, 0))
                   for li in range(3)],
        out_shape=[jax.ShapeDtypeStruct((n_list[li], cs[li]), _f32)
                   for li in range(3)],
        interpret=_INTERPRET,
    )(*gcn_in)

    outs = []
    for li, (c, hw) in enumerate(_LEVELS):
        HW = hw * hw
        ns = ns_list[li]
        out = pl.pallas_call(
            functools.partial(_scatter_t_body, ns),
            grid=(bsz,),
            in_specs=[_batch_spec((HW, c)), _batch_spec((ns, c)),
                      _batch_spec((1, ns), pltpu.SMEM)],
            out_specs=_batch_spec((c, HW)),
            out_shape=jax.ShapeDtypeStruct((bsz, c, HW), _f32),
            scratch_shapes=[pltpu.VMEM((HW, c), _f32)],
            interpret=_INTERPRET,
        )(fuseds[li], us[li].reshape(bsz, ns, c), idxs[li])
        outs.append(out.reshape(bsz, c, hw, hw))
    return tuple(outs)


# speedup vs baseline: 1.2213x; 1.2213x over previous
import functools
import math

import jax
import jax.numpy as jnp
from jax import lax
from jax.experimental import pallas as pl
from jax.experimental.pallas import tpu as pltpu

_NC = 80
_C_TEXT = 512
_EMBED = 128
_GCN_H = 64
_THR = 0.5
_LEVELS = [(192, 64), (384, 32), (768, 16)]
_K_RATIO = 0.005
_INTERPRET = False

_f32 = jnp.float32


def _silu(x):
    return x * jax.nn.sigmoid(x)


def _conv3(xin, xp_ref, w_ref, W, HW):
    cin = xin.shape[1]
    xp_ref[0:W + 1, :] = jnp.zeros((W + 1, cin), _f32)
    xp_ref[W + 1:W + 1 + HW, :] = xin
    xp_ref[W + 1 + HW:, :] = jnp.zeros((W + 1, cin), _f32)
    col = lax.broadcasted_iota(jnp.int32, (HW, 1), 0) & (W - 1)
    cout = w_ref.shape[3]
    acc = jnp.zeros((HW, cout), _f32)
    for dy in range(3):
        for dx in range(3):
            off = dy * W + dx
            tap = xp_ref[off:off + HW, :]
            if dx == 0:
                tap = jnp.where(col != 0, tap, 0.0)
            elif dx == 2:
                tap = jnp.where(col != W - 1, tap, 0.0)
            acc = acc + jax.lax.dot(tap, w_ref[dy, dx],
                                    preferred_element_type=_f32)
    return acc


def _trunk_body(H, W, ch, hid, c, ns,
                p1_ref, p2_ref, text_ref,
                wcv1_ref, bcv1_ref, wcv2_ref, bcv2_ref,
                wimg_ref, bimg_ref, twt_ref, tb_ref, ab_ref,
                wproj_ref, bproj_ref, wf_ref, bf_ref,
                fused_ref, top_ref, idx_ref,
                xp1_ref, xp2_ref):
    HW = H * W
    p1 = p1_ref[0]
    p2 = p2_ref[0]
    hidv = _silu(jax.lax.dot(p2, wcv1_ref[...], preferred_element_type=_f32)
                 + bcv1_ref[...])
    bout = p2 + _silu(_conv3(hidv, xp1_ref, wcv2_ref, W, HW) + bcv2_ref[...])
    ie = jax.lax.dot(bout, wimg_ref[...], preferred_element_type=_f32) \
        + bimg_ref[...]
    t = jax.lax.dot(text_ref[0], twt_ref[...], preferred_element_type=_f32) \
        + tb_ref[...]
    aw = lax.dot_general(ie, t, (((1,), (1,)), ((), ())),
                         preferred_element_type=_f32)
    s = jax.nn.sigmoid(jnp.max(aw, axis=1, keepdims=True)
                       * (1.0 / math.sqrt(_EMBED)) + ab_ref[...])
    awt = lax.dot_general(t, ie, (((1,), (1,)), ((), ())),
                          preferred_element_type=_f32)
    s_row = jax.nn.sigmoid(jnp.max(awt, axis=0, keepdims=True)
                           * (1.0 / math.sqrt(_EMBED)) + ab_ref[...])
    attn = (_conv3(bout, xp2_ref, wproj_ref, W, HW) + bproj_ref[...]) * s
    fused = (jax.lax.dot(p1, wf_ref[0:ch, :], preferred_element_type=_f32)
             + jax.lax.dot(p2, wf_ref[ch:2 * ch, :],
                           preferred_element_type=_f32)
             + jax.lax.dot(bout, wf_ref[2 * ch:3 * ch, :],
                           preferred_element_type=_f32)
             + jax.lax.dot(attn, wf_ref[3 * ch:4 * ch, :],
                           preferred_element_type=_f32)
             + bf_ref[...])
    fused_ref[0] = fused
    lane = lax.broadcasted_iota(jnp.int32, (1, HW), 1)
    lane_ns = lax.broadcasted_iota(jnp.int32, (1, ns), 1)
    sub = lax.broadcasted_iota(jnp.int32, (ns, 1), 0)
    sc = s_row
    r = jnp.zeros((1, HW), jnp.int32)
    idxs = jnp.zeros((1, ns), jnp.int32)
    for j in range(ns):
        m = jnp.max(sc, axis=1, keepdims=True)
        cand = jnp.where(sc == m, lane, HW)
        ix = jnp.min(cand, axis=1, keepdims=True)
        hit = (lane == ix)
        r = r + hit.astype(jnp.int32) * (j + 1)
        idxs = idxs + jnp.where(lane_ns == j, ix, 0)
        sc = jnp.where(hit, -jnp.inf, sc)
    sel = (r == sub + 1).astype(_f32)
    top_ref[0] = jax.lax.dot(sel, fused, preferred_element_type=_f32)
    idx_ref[0] = idxs


def _gcn_body(n, top_ref, w1_ref, b1_ref, w2_ref, b2_ref, u_ref):
    top = top_ref[...]
    nrm = jnp.maximum(jnp.sqrt(jnp.sum(top * top, axis=1, keepdims=True)),
                      1e-12)
    nf = top / nrm
    sim = lax.dot_general(nf, nf, (((1,), (1,)), ((), ())),
                          preferred_element_type=_f32)
    ri = lax.broadcasted_iota(jnp.int32, (n, n), 0)
    ci = lax.broadcasted_iota(jnp.int32, (n, n), 1)
    at = (sim > _THR).astype(_f32) + (ri == ci).astype(_f32)
    degr = jnp.sum(at, axis=1, keepdims=True)
    degc = jnp.sum(at, axis=0, keepdims=True)
    dr = jnp.where(degr > 0, 1.0 / jnp.sqrt(degr), 0.0)
    dc = jnp.where(degc > 0, 1.0 / jnp.sqrt(degc), 0.0)
    p = at * dr * dc
    h1 = jax.nn.relu(
        jax.lax.dot(p, jax.lax.dot(top, w1_ref[...],
                                   preferred_element_type=_f32),
                    preferred_element_type=_f32) + b1_ref[...])
    u = jax.lax.dot(p, jax.lax.dot(h1, w2_ref[...],
                                   preferred_element_type=_f32),
                    preferred_element_type=_f32) + b2_ref[...]
    u_ref[...] = u


def _scatter_t_body(ns, fused_ref, u_ref, idx_ref, out_ref, xp_ref):
    xp_ref[...] = fused_ref[0]
    for j in range(ns):
        xp_ref[pl.ds(idx_ref[0, 0, j], 1), :] = u_ref[0, j:j + 1, :]
    out_ref[0] = xp_ref[...].T


def _batch_spec(shape, space=None):
    if space is None:
        return pl.BlockSpec((1,) + tuple(shape),
                            lambda b: (b,) + (0,) * len(shape))
    return pl.BlockSpec((1,) + tuple(shape),
                        lambda b: (b,) + (0,) * len(shape),
                        memory_space=space)


def _full_spec(a):
    nd = a.ndim
    return pl.BlockSpec(a.shape, lambda b, _n=nd: (0,) * _n)


def _fold1x1(p):
    w = (p['w'][:, :, 0, 0] * p['gamma'][:, None]).T
    return w, p['beta'][None, :]


def _fold3x3(p):
    w = (p['w'] * p['gamma'][:, None, None, None]).transpose(2, 3, 1, 0)
    return w, p['beta'][None, :]


def _level(xr, text_feat, p, c, hw_side, ns):
    bsz = xr.shape[0]
    H = W = hw_side
    HW = H * W
    ch = c // 2
    hid = ch // 2
    n = bsz * ns

    p1 = xr[:, :, :ch]
    p2 = xr[:, :, ch:]
    wcv1, bcv1 = _fold1x1(p['cv1'])
    wcv2, bcv2 = _fold3x3(p['cv2'])
    wimg, bimg = _fold1x1(p['img_conv'])
    twt = p['text_w'].T
    tb = p['text_b'][None, :]
    ab = p['attn_bias'].reshape(1, 1)
    wproj, bproj = _fold3x3(p['proj_conv'])
    wf, bf = _fold1x1(p['fusion'])

    trunk_in = (p1, p2, text_feat, wcv1, bcv1, wcv2, bcv2, wimg, bimg,
                twt, tb, ab, wproj, bproj, wf, bf)
    in_specs = [_batch_spec((HW, ch)), _batch_spec((HW, ch)),
                _batch_spec((_NC, _C_TEXT))] + \
        [_full_spec(a) for a in trunk_in[3:]]
    fused, top, idx = pl.pallas_call(
        functools.partial(_trunk_body, H, W, ch, hid, c, ns),
        grid=(bsz,),
        in_specs=in_specs,
        out_specs=[_batch_spec((HW, c)), _batch_spec((ns, c)),
                   _batch_spec((1, ns))],
        out_shape=[jax.ShapeDtypeStruct((bsz, HW, c), _f32),
                   jax.ShapeDtypeStruct((bsz, ns, c), _f32),
                   jax.ShapeDtypeStruct((bsz, 1, ns), jnp.int32)],
        scratch_shapes=[pltpu.VMEM((HW + 2 * W + 2, hid), _f32),
                        pltpu.VMEM((HW + 2 * W + 2, ch), _f32)],
        interpret=_INTERPRET,
    )(*trunk_in)

    topf = top.reshape(n, c)
    gcn_in = (topf, p['gcn1_w'], p['gcn1_b'][None, :],
              p['gcn2_w'], p['gcn2_b'][None, :])
    u = pl.pallas_call(
        functools.partial(_gcn_body, n),
        in_specs=[pl.BlockSpec(a.shape, functools.partial(
            lambda _n: (0,) * _n, a.ndim)) for a in gcn_in],
        out_specs=pl.BlockSpec((n, c), lambda: (0, 0)),
        out_shape=jax.ShapeDtypeStruct((n, c), _f32),
        interpret=_INTERPRET,
    )(*gcn_in)

    out = pl.pallas_call(
        functools.partial(_scatter_t_body, ns),
        grid=(bsz,),
        in_specs=[_batch_spec((HW, c)), _batch_spec((ns, c)),
                  _batch_spec((1, ns), pltpu.SMEM)],
        out_specs=_batch_spec((c, HW)),
        out_shape=jax.ShapeDtypeStruct((bsz, c, HW), _f32),
        scratch_shapes=[pltpu.VMEM((HW, c), _f32)],
        interpret=_INTERPRET,
    )(fused, u.reshape(bsz, ns, c), idx)
    return out.reshape(bsz, c, H, W)


def kernel(x3, x4, x5, text_feat, params):
    outs = []
    for i, (x, (c, hw)) in enumerate(zip((x3, x4, x5), _LEVELS)):
        ns = int(hw * hw * _K_RATIO)
        bsz = x.shape[0]
        xr = x.reshape(bsz, c, hw * hw).transpose(0, 2, 1)
        outs.append(_level(xr, text_feat, params['l%d' % i], c, hw, ns))
    return tuple(outs)
